# Initial kernel scaffold; baseline (speedup 1.0000x reference)
import functools

import jax
import jax.numpy as jnp
from jax import lax
from jax.experimental import pallas as pl
from jax.experimental.pallas import tpu as pltpu
from jax.experimental.pallas import tpu_sc as plsc

N = 10000
E = 320000
D_IN = 128
D_HID = 128
D_OUT = 64

NC = 2
NS = 16
NW = NC * NS
EPW = E // NW
C = 80
NCH = EPW // C
RPT = N // NS
ZR = 125


def _make_agg(with_deg: bool):
    out_type = [jax.ShapeDtypeStruct((NC, N, D_HID), jnp.float32)]
    scratch = [
        pltpu.VMEM((NCH, C), jnp.int32),
        pltpu.VMEM((NCH, C), jnp.int32),
        pltpu.VMEM((C, D_HID), jnp.float32),
        pltpu.VMEM((ZR, D_HID), jnp.float32),
        pltpu.VMEM_SHARED((N, D_HID), jnp.float32),
        pltpu.SemaphoreType.DMA,
    ]
    if with_deg:
        out_type.append(jax.ShapeDtypeStruct((NC, N, 16), jnp.float32))
        scratch += [
            pltpu.VMEM((C, 16), jnp.float32),
            pltpu.VMEM((RPT, 16), jnp.float32),
            pltpu.VMEM_SHARED((N, 16), jnp.float32),
        ]

    def body(h_hbm, src_hbm, dst_hbm, *rest):
        if with_deg:
            (part_hbm, deg_hbm, src_v, dst_v, rows_v, zrow_v, acc_sh, sem,
             ones_v, zdeg_v, deg_sh) = rest
        else:
            (part_hbm, src_v, dst_v, rows_v, zrow_v, acc_sh, sem) = rest
        cid = lax.axis_index("c")
        sid = lax.axis_index("s")
        w = cid * NS + sid

        zero16 = jnp.zeros((16,), jnp.float32)

        def zr_body(r, carry):
            for jj in range(D_HID // 16):
                zrow_v[r, pl.ds(jj * 16, 16)] = zero16
            return carry
        lax.fori_loop(0, ZR, zr_body, 0)
        for t in range(RPT // ZR):
            pltpu.sync_copy(zrow_v, acc_sh.at[pl.ds(sid * RPT + t * ZR, ZR)])

        if with_deg:
            def zd_body(r, carry):
                zdeg_v[r, pl.ds(0, 16)] = zero16
                return carry
            lax.fori_loop(0, RPT, zd_body, 0)

            one16 = jnp.ones((16,), jnp.float32)

            def on_body(r, carry):
                ones_v[r, pl.ds(0, 16)] = one16
                return carry
            lax.fori_loop(0, C, on_body, 0)
            pltpu.sync_copy(zdeg_v, deg_sh.at[pl.ds(sid * RPT, RPT)])

        plsc.subcore_barrier()

        pltpu.sync_copy(src_hbm.at[w], src_v)
        pltpu.sync_copy(dst_hbm.at[w], dst_v)

        def chunk(j, carry):
            pltpu.async_copy(h_hbm.at[src_v.at[j]], rows_v, sem).wait()
            pltpu.sync_copy(rows_v, acc_sh.at[dst_v.at[j]], add=True)
            if with_deg:
                pltpu.sync_copy(ones_v, deg_sh.at[dst_v.at[j]], add=True)
            return carry
        lax.fori_loop(0, NCH, chunk, 0)

        plsc.subcore_barrier()

        pltpu.sync_copy(acc_sh.at[pl.ds(sid * RPT, RPT)],
                        part_hbm.at[cid, pl.ds(sid * RPT, RPT)])
        if with_deg:
            pltpu.sync_copy(deg_sh.at[pl.ds(sid * RPT, RPT)],
                            deg_hbm.at[cid, pl.ds(sid * RPT, RPT)])

    mesh = plsc.VectorSubcoreMesh(core_axis_name="c", subcore_axis_name="s")
    return pl.kernel(body, out_type=out_type, mesh=mesh,
                     scratch_types=scratch)


_agg_deg = _make_agg(with_deg=True)
_agg = _make_agg(with_deg=False)


BR = 1000


def _fe_body(x_ref, w_ref, b_ref, o_ref):
    z = jnp.dot(x_ref[...], w_ref[...], preferred_element_type=jnp.float32)
    o_ref[...] = jnp.tanh(z + b_ref[...])


def _fe(x, W, b):
    return pl.pallas_call(
        _fe_body,
        grid=(N // BR,),
        in_specs=[
            pl.BlockSpec((BR, D_IN), lambda i: (i, 0)),
            pl.BlockSpec((D_IN, D_HID), lambda i: (0, 0)),
            pl.BlockSpec((1, D_HID), lambda i: (0, 0)),
        ],
        out_specs=pl.BlockSpec((BR, D_HID), lambda i: (i, 0)),
        out_shape=jax.ShapeDtypeStruct((N, D_HID), jnp.float32),
    )(x, W, b.reshape(1, D_HID))


def _mix_body(h_ref, p0_ref, p1_ref, dg_ref, w_ref, b_ref, o_ref):
    dsum = jnp.maximum(dg_ref[:, 0:1] + dg_ref[:, 1:2], 1.0)
    agg = (p0_ref[...] + p1_ref[...]) / dsum
    z = jnp.dot(h_ref[...] + agg, w_ref[...],
                preferred_element_type=jnp.float32)
    o_ref[...] = jnp.maximum(z + b_ref[...], 0.0)


def _mix(h, p0, p1, degT, W, b):
    return pl.pallas_call(
        _mix_body,
        grid=(N // BR,),
        in_specs=[
            pl.BlockSpec((BR, D_HID), lambda i: (i, 0)),
            pl.BlockSpec((BR, D_HID), lambda i: (i, 0)),
            pl.BlockSpec((BR, D_HID), lambda i: (i, 0)),
            pl.BlockSpec((BR, 2), lambda i: (i, 0)),
            pl.BlockSpec((D_HID, D_HID), lambda i: (0, 0)),
            pl.BlockSpec((1, D_HID), lambda i: (0, 0)),
        ],
        out_specs=pl.BlockSpec((BR, D_HID), lambda i: (i, 0)),
        out_shape=jax.ShapeDtypeStruct((N, D_HID), jnp.float32),
    )(h, p0, p1, degT, W, b.reshape(1, D_HID))


def _mix_fc_body(h_ref, p0_ref, p1_ref, dg_ref, w_ref, b_ref,
                 wfc_ref, bfc_ref, o_ref):
    dsum = jnp.maximum(dg_ref[:, 0:1] + dg_ref[:, 1:2], 1.0)
    agg = (p0_ref[...] + p1_ref[...]) / dsum
    z = jnp.dot(h_ref[...] + agg, w_ref[...],
                preferred_element_type=jnp.float32)
    h2 = jnp.maximum(z + b_ref[...], 0.0)
    o_ref[...] = jnp.dot(h2, wfc_ref[...],
                         preferred_element_type=jnp.float32) + bfc_ref[...]


def _mix_fc(h, p0, p1, degT, W, b, Wfc, bfc):
    return pl.pallas_call(
        _mix_fc_body,
        grid=(N // BR,),
        in_specs=[
            pl.BlockSpec((BR, D_HID), lambda i: (i, 0)),
            pl.BlockSpec((BR, D_HID), lambda i: (i, 0)),
            pl.BlockSpec((BR, D_HID), lambda i: (i, 0)),
            pl.BlockSpec((BR, 2), lambda i: (i, 0)),
            pl.BlockSpec((D_HID, D_HID), lambda i: (0, 0)),
            pl.BlockSpec((1, D_HID), lambda i: (0, 0)),
            pl.BlockSpec((D_HID, D_OUT), lambda i: (0, 0)),
            pl.BlockSpec((1, D_OUT), lambda i: (0, 0)),
        ],
        out_specs=pl.BlockSpec((BR, D_OUT), lambda i: (i, 0)),
        out_shape=jax.ShapeDtypeStruct((N, D_OUT), jnp.float32),
    )(h, p0, p1, degT, W, b.reshape(1, D_HID), Wfc, bfc.reshape(1, D_OUT))


def kernel(x, edge_index, W_fe, b_fe, W_g1, b_g1, W_g2, b_g2, W_fc, b_fc):
    src = edge_index[0].astype(jnp.int32).reshape(NW, NCH, C)
    dst = edge_index[1].astype(jnp.int32).reshape(NW, NCH, C)

    h0 = _fe(x, W_fe, b_fe)
    part1, deg16 = _agg_deg(h0, src, dst)
    degT = deg16[:, :, 0].T

    h1 = _mix(h0, part1[0], part1[1], degT, W_g1, b_g1)
    part2 = _agg(h1, src, dst)
    return _mix_fc(h1, part2[0], part2[1], degT, W_g2, b_g2, W_fc, b_fc)


# speedup vs baseline: 4.0896x; 4.0896x over previous
import jax
import jax.numpy as jnp
from jax import lax
from jax.experimental import pallas as pl
from jax.experimental.pallas import tpu as pltpu
from jax.experimental.pallas import tpu_sc as plsc

N = 10000
E = 320000
D_IN = 128
D_HID = 128
D_OUT = 64

NC = 2
NS = 16
NW = NC * NS
EPW = E // NW
C = 80
NCH = EPW // C
STRIDE = 624
WIN = 640
ZR = 16


def _agg_body(h_hbm, src_hbm, dst_hbm, part_hbm,
              src_v, dst_v, rows_v, zrow_v, acc_sh, sem):
    cid = lax.axis_index("c")
    sid = lax.axis_index("s")
    w = cid * NS + sid

    zero16 = jnp.zeros((16,), jnp.float32)

    def zr_body(r, carry):
        for jj in range(D_HID // 16):
            zrow_v[r, pl.ds(jj * 16, 16)] = zero16
        return carry
    lax.fori_loop(0, ZR, zr_body, 0)
    for t in range(WIN // ZR):
        pltpu.sync_copy(zrow_v, acc_sh.at[pl.ds(sid * STRIDE + t * ZR, ZR)])

    plsc.subcore_barrier()

    def chunk(j, carry):
        pltpu.sync_copy(src_hbm.at[w, j], src_v)
        pltpu.sync_copy(dst_hbm.at[w, j], dst_v)
        pltpu.async_copy(h_hbm.at[src_v], rows_v, sem).wait()
        pltpu.sync_copy(rows_v, acc_sh.at[dst_v], add=True)
        return carry
    lax.fori_loop(0, NCH, chunk, 0)

    plsc.subcore_barrier()

    pltpu.sync_copy(acc_sh.at[pl.ds(sid * STRIDE, WIN)],
                    part_hbm.at[cid, pl.ds(sid * STRIDE, WIN)])


def _deg_body(dst_hbm, deg_hbm, dst_v, ones_v, zrow_v, deg_sh):
    cid = lax.axis_index("c")
    sid = lax.axis_index("s")
    w = cid * NS + sid

    zero16 = jnp.zeros((16,), jnp.float32)
    one16 = jnp.ones((16,), jnp.float32)

    def zr_body(r, carry):
        for jj in range(D_HID // 16):
            zrow_v[r, pl.ds(jj * 16, 16)] = zero16
        return carry
    lax.fori_loop(0, ZR, zr_body, 0)

    def on_body(r, carry):
        for jj in range(D_HID // 16):
            ones_v[r, pl.ds(jj * 16, 16)] = one16
        return carry
    lax.fori_loop(0, C, on_body, 0)

    for t in range(WIN // ZR):
        pltpu.sync_copy(zrow_v, deg_sh.at[pl.ds(sid * STRIDE + t * ZR, ZR)])

    plsc.subcore_barrier()

    def chunk(j, carry):
        pltpu.sync_copy(dst_hbm.at[w, j], dst_v)
        pltpu.sync_copy(ones_v, deg_sh.at[dst_v], add=True)
        return carry
    lax.fori_loop(0, NCH, chunk, 0)

    plsc.subcore_barrier()

    pltpu.sync_copy(deg_sh.at[pl.ds(sid * STRIDE, WIN)],
                    deg_hbm.at[cid, pl.ds(sid * STRIDE, WIN)])


_SC_MESH = plsc.VectorSubcoreMesh(core_axis_name="c", subcore_axis_name="s")

_agg = pl.kernel(
    _agg_body,
    out_type=[jax.ShapeDtypeStruct((NC, N, D_HID), jnp.float32)],
    mesh=_SC_MESH,
    scratch_types=[
        pltpu.VMEM((C,), jnp.int32),
        pltpu.VMEM((C,), jnp.int32),
        pltpu.VMEM((C, D_HID), jnp.float32),
        pltpu.VMEM((ZR, D_HID), jnp.float32),
        pltpu.VMEM_SHARED((N, D_HID), jnp.float32),
        pltpu.SemaphoreType.DMA,
    ],
)

_deg = pl.kernel(
    _deg_body,
    out_type=[jax.ShapeDtypeStruct((NC, N, D_HID), jnp.float32)],
    mesh=_SC_MESH,
    scratch_types=[
        pltpu.VMEM((C,), jnp.int32),
        pltpu.VMEM((C, D_HID), jnp.float32),
        pltpu.VMEM((ZR, D_HID), jnp.float32),
        pltpu.VMEM_SHARED((N, D_HID), jnp.float32),
    ],
)


BR = 1000


def _fe_body(x_ref, w_ref, b_ref, o_ref):
    z = jnp.dot(x_ref[...], w_ref[...], preferred_element_type=jnp.float32)
    o_ref[...] = jnp.tanh(z + b_ref[...])


def _fe(x, W, b):
    return pl.pallas_call(
        _fe_body,
        grid=(N // BR,),
        in_specs=[
            pl.BlockSpec((BR, D_IN), lambda i: (i, 0)),
            pl.BlockSpec((D_IN, D_HID), lambda i: (0, 0)),
            pl.BlockSpec((1, D_HID), lambda i: (0, 0)),
        ],
        out_specs=pl.BlockSpec((BR, D_HID), lambda i: (i, 0)),
        out_shape=jax.ShapeDtypeStruct((N, D_HID), jnp.float32),
    )(x, W, b.reshape(1, D_HID))


def _mix_body(h_ref, p0_ref, p1_ref, d0_ref, d1_ref, w_ref, b_ref, o_ref):
    dsum = jnp.maximum(d0_ref[:, 0:1] + d1_ref[:, 0:1], 1.0)
    agg = (p0_ref[...] + p1_ref[...]) / dsum
    z = jnp.dot(h_ref[...] + agg, w_ref[...],
                preferred_element_type=jnp.float32)
    o_ref[...] = jnp.maximum(z + b_ref[...], 0.0)


def _mix(h, p0, p1, d0, d1, W, b):
    return pl.pallas_call(
        _mix_body,
        grid=(N // BR,),
        in_specs=[
            pl.BlockSpec((BR, D_HID), lambda i: (i, 0)),
            pl.BlockSpec((BR, D_HID), lambda i: (i, 0)),
            pl.BlockSpec((BR, D_HID), lambda i: (i, 0)),
            pl.BlockSpec((BR, D_HID), lambda i: (i, 0)),
            pl.BlockSpec((BR, D_HID), lambda i: (i, 0)),
            pl.BlockSpec((D_HID, D_HID), lambda i: (0, 0)),
            pl.BlockSpec((1, D_HID), lambda i: (0, 0)),
        ],
        out_specs=pl.BlockSpec((BR, D_HID), lambda i: (i, 0)),
        out_shape=jax.ShapeDtypeStruct((N, D_HID), jnp.float32),
    )(h, p0, p1, d0, d1, W, b.reshape(1, D_HID))


def _mix_fc_body(h_ref, p0_ref, p1_ref, d0_ref, d1_ref, w_ref, b_ref,
                 wfc_ref, bfc_ref, o_ref):
    dsum = jnp.maximum(d0_ref[:, 0:1] + d1_ref[:, 0:1], 1.0)
    agg = (p0_ref[...] + p1_ref[...]) / dsum
    z = jnp.dot(h_ref[...] + agg, w_ref[...],
                preferred_element_type=jnp.float32)
    h2 = jnp.maximum(z + b_ref[...], 0.0)
    o_ref[...] = jnp.dot(h2, wfc_ref[...],
                         preferred_element_type=jnp.float32) + bfc_ref[...]


def _mix_fc(h, p0, p1, d0, d1, W, b, Wfc, bfc):
    return pl.pallas_call(
        _mix_fc_body,
        grid=(N // BR,),
        in_specs=[
            pl.BlockSpec((BR, D_HID), lambda i: (i, 0)),
            pl.BlockSpec((BR, D_HID), lambda i: (i, 0)),
            pl.BlockSpec((BR, D_HID), lambda i: (i, 0)),
            pl.BlockSpec((BR, D_HID), lambda i: (i, 0)),
            pl.BlockSpec((BR, D_HID), lambda i: (i, 0)),
            pl.BlockSpec((D_HID, D_HID), lambda i: (0, 0)),
            pl.BlockSpec((1, D_HID), lambda i: (0, 0)),
            pl.BlockSpec((D_HID, D_OUT), lambda i: (0, 0)),
            pl.BlockSpec((1, D_OUT), lambda i: (0, 0)),
        ],
        out_specs=pl.BlockSpec((BR, D_OUT), lambda i: (i, 0)),
        out_shape=jax.ShapeDtypeStruct((N, D_OUT), jnp.float32),
    )(h, p0, p1, d0, d1, W, b.reshape(1, D_HID), Wfc, bfc.reshape(1, D_OUT))


def kernel(x, edge_index, W_fe, b_fe, W_g1, b_g1, W_g2, b_g2, W_fc, b_fc):
    src = edge_index[0].astype(jnp.int32).reshape(NW, NCH, C)
    dst = edge_index[1].astype(jnp.int32).reshape(NW, NCH, C)

    h0 = _fe(x, W_fe, b_fe)
    (degp,) = _deg(dst)
    (p1,) = _agg(h0, src, dst)
    h1 = _mix(h0, p1[0], p1[1], degp[0], degp[1], W_g1, b_g1)
    (p2,) = _agg(h1, src, dst)
    return _mix_fc(h1, p2[0], p2[1], degp[0], degp[1], W_g2, b_g2, W_fc, b_fc)


# speedup vs baseline: 7.6896x; 1.8803x over previous
import jax
import jax.numpy as jnp
from jax import lax
from jax.experimental import pallas as pl
from jax.experimental.pallas import tpu as pltpu
from jax.experimental.pallas import tpu_sc as plsc

N = 10000
E = 320000
D_IN = 128
D_HID = 128
D_OUT = 64

NC = 2
NS = 16
NW = NC * NS
EPW = E // NW
C = 96
NCHF = 104
CT = EPW - NCHF * C
STRIDE = 624
WIN = 640
ZR = 32


def _zero_rows(zrow_v, tgt_sh, sid):
    zero16 = jnp.zeros((16,), jnp.float32)

    def zr_body(r, carry):
        for jj in range(D_HID // 16):
            zrow_v[r, pl.ds(jj * 16, 16)] = zero16
        return carry
    lax.fori_loop(0, ZR, zr_body, 0)
    for t in range(WIN // ZR):
        pltpu.sync_copy(zrow_v, tgt_sh.at[pl.ds(sid * STRIDE + t * ZR, ZR)])


def _agg_body(h_hbm, srcf_hbm, dstf_hbm, srct_hbm, dstt_hbm, part_hbm,
              srcb_v, dstb_v, rows_v, srct_v, dstt_v, rowst_v, zrow_v,
              acc_sh, sg0, sg1, si0, si1):
    cid = lax.axis_index("c")
    sid = lax.axis_index("s")
    w = cid * NS + sid
    sg = (sg0, sg1)
    si = (si0, si1)

    _zero_rows(zrow_v, acc_sh, sid)
    plsc.subcore_barrier()

    pltpu.sync_copy(srcf_hbm.at[w, 0], srcb_v.at[0])
    pltpu.sync_copy(dstf_hbm.at[w, 0], dstb_v.at[0])
    pltpu.async_copy(h_hbm.at[srcb_v.at[0]], rows_v.at[0], sg[0])
    pltpu.async_copy(srcf_hbm.at[w, 1], srcb_v.at[1], si[1])
    pltpu.async_copy(dstf_hbm.at[w, 1], dstb_v.at[1], si[1])

    def outer(jj, carry):
        for b in range(2):
            ob = 1 - b
            j = 2 * jj + b
            j1 = jnp.minimum(j + 1, NCHF - 1)
            j2 = jnp.minimum(j + 2, NCHF - 1)
            pltpu.make_async_copy(h_hbm.at[srcb_v.at[b]], rows_v.at[b],
                                  sg[b]).wait()
            pltpu.make_async_copy(srcf_hbm.at[w, j1], srcb_v.at[ob],
                                  si[ob]).wait()
            pltpu.make_async_copy(dstf_hbm.at[w, j1], dstb_v.at[ob],
                                  si[ob]).wait()
            pltpu.async_copy(h_hbm.at[srcb_v.at[ob]], rows_v.at[ob], sg[ob])
            pltpu.sync_copy(rows_v.at[b], acc_sh.at[dstb_v.at[b]], add=True)
            pltpu.async_copy(srcf_hbm.at[w, j2], srcb_v.at[b], si[b])
            pltpu.async_copy(dstf_hbm.at[w, j2], dstb_v.at[b], si[b])
        return carry
    lax.fori_loop(0, NCHF // 2, outer, 0)

    pltpu.make_async_copy(h_hbm.at[srcb_v.at[0]], rows_v.at[0], sg[0]).wait()
    pltpu.make_async_copy(srcf_hbm.at[w, NCHF - 1], srcb_v.at[1],
                          si[1]).wait()
    pltpu.make_async_copy(dstf_hbm.at[w, NCHF - 1], dstb_v.at[1],
                          si[1]).wait()

    pltpu.sync_copy(srct_hbm.at[w], srct_v)
    pltpu.sync_copy(dstt_hbm.at[w], dstt_v)
    pltpu.async_copy(h_hbm.at[srct_v], rowst_v, sg[0]).wait()
    pltpu.sync_copy(rowst_v, acc_sh.at[dstt_v], add=True)

    plsc.subcore_barrier()

    pltpu.sync_copy(acc_sh.at[pl.ds(sid * STRIDE, WIN)],
                    part_hbm.at[cid, pl.ds(sid * STRIDE, WIN)])


def _deg_body(dstf_hbm, dstt_hbm, deg_hbm, dstb_v, dstt_v, ones_v, zrow_v,
              deg_sh, si0, si1):
    cid = lax.axis_index("c")
    sid = lax.axis_index("s")
    w = cid * NS + sid
    si = (si0, si1)

    one16 = jnp.ones((16,), jnp.float32)

    def on_body(r, carry):
        for jj in range(D_HID // 16):
            ones_v[r, pl.ds(jj * 16, 16)] = one16
        return carry
    lax.fori_loop(0, C, on_body, 0)

    _zero_rows(zrow_v, deg_sh, sid)
    plsc.subcore_barrier()

    pltpu.sync_copy(dstf_hbm.at[w, 0], dstb_v.at[0])
    pltpu.async_copy(dstf_hbm.at[w, 1], dstb_v.at[1], si[1])

    def outer(jj, carry):
        for b in range(2):
            ob = 1 - b
            j = 2 * jj + b
            j1 = jnp.minimum(j + 1, NCHF - 1)
            pltpu.sync_copy(ones_v, deg_sh.at[dstb_v.at[b]], add=True)
            pltpu.make_async_copy(dstf_hbm.at[w, j1], dstb_v.at[ob],
                                  si[ob]).wait()
            pltpu.async_copy(dstf_hbm.at[w, jnp.minimum(j + 2, NCHF - 1)],
                             dstb_v.at[b], si[b])
        return carry
    lax.fori_loop(0, NCHF // 2, outer, 0)

    pltpu.make_async_copy(dstf_hbm.at[w, NCHF - 1], dstb_v.at[1],
                          si[1]).wait()

    pltpu.sync_copy(dstt_hbm.at[w], dstt_v)
    pltpu.sync_copy(ones_v.at[pl.ds(0, CT)], deg_sh.at[dstt_v], add=True)

    plsc.subcore_barrier()

    pltpu.sync_copy(deg_sh.at[pl.ds(sid * STRIDE, WIN)],
                    deg_hbm.at[cid, pl.ds(sid * STRIDE, WIN)])


_SC_MESH = plsc.VectorSubcoreMesh(core_axis_name="c", subcore_axis_name="s")

_agg = pl.kernel(
    _agg_body,
    out_type=[jax.ShapeDtypeStruct((NC, N, D_HID), jnp.float32)],
    mesh=_SC_MESH,
    scratch_types=[
        pltpu.VMEM((2, C), jnp.int32),
        pltpu.VMEM((2, C), jnp.int32),
        pltpu.VMEM((2, C, D_HID), jnp.float32),
        pltpu.VMEM((CT,), jnp.int32),
        pltpu.VMEM((CT,), jnp.int32),
        pltpu.VMEM((CT, D_HID), jnp.float32),
        pltpu.VMEM((ZR, D_HID), jnp.float32),
        pltpu.VMEM_SHARED((N, D_HID), jnp.float32),
        pltpu.SemaphoreType.DMA,
        pltpu.SemaphoreType.DMA,
        pltpu.SemaphoreType.DMA,
        pltpu.SemaphoreType.DMA,
    ],
)

_deg = pl.kernel(
    _deg_body,
    out_type=[jax.ShapeDtypeStruct((NC, N, D_HID), jnp.float32)],
    mesh=_SC_MESH,
    scratch_types=[
        pltpu.VMEM((2, C), jnp.int32),
        pltpu.VMEM((CT,), jnp.int32),
        pltpu.VMEM((C, D_HID), jnp.float32),
        pltpu.VMEM((ZR, D_HID), jnp.float32),
        pltpu.VMEM_SHARED((N, D_HID), jnp.float32),
        pltpu.SemaphoreType.DMA,
        pltpu.SemaphoreType.DMA,
    ],
)


BR = 1000


def _fe_body(x_ref, w_ref, b_ref, o_ref):
    z = jnp.dot(x_ref[...], w_ref[...], preferred_element_type=jnp.float32)
    o_ref[...] = jnp.tanh(z + b_ref[...])


def _fe(x, W, b):
    return pl.pallas_call(
        _fe_body,
        grid=(N // BR,),
        in_specs=[
            pl.BlockSpec((BR, D_IN), lambda i: (i, 0)),
            pl.BlockSpec((D_IN, D_HID), lambda i: (0, 0)),
            pl.BlockSpec((1, D_HID), lambda i: (0, 0)),
        ],
        out_specs=pl.BlockSpec((BR, D_HID), lambda i: (i, 0)),
        out_shape=jax.ShapeDtypeStruct((N, D_HID), jnp.float32),
    )(x, W, b.reshape(1, D_HID))


def _mix_body(h_ref, p0_ref, p1_ref, d0_ref, d1_ref, w_ref, b_ref, o_ref):
    dsum = jnp.maximum(d0_ref[:, 0:1] + d1_ref[:, 0:1], 1.0)
    agg = (p0_ref[...] + p1_ref[...]) / dsum
    z = jnp.dot(h_ref[...] + agg, w_ref[...],
                preferred_element_type=jnp.float32)
    o_ref[...] = jnp.maximum(z + b_ref[...], 0.0)


def _mix(h, p0, p1, d0, d1, W, b):
    return pl.pallas_call(
        _mix_body,
        grid=(N // BR,),
        in_specs=[
            pl.BlockSpec((BR, D_HID), lambda i: (i, 0)),
            pl.BlockSpec((BR, D_HID), lambda i: (i, 0)),
            pl.BlockSpec((BR, D_HID), lambda i: (i, 0)),
            pl.BlockSpec((BR, D_HID), lambda i: (i, 0)),
            pl.BlockSpec((BR, D_HID), lambda i: (i, 0)),
            pl.BlockSpec((D_HID, D_HID), lambda i: (0, 0)),
            pl.BlockSpec((1, D_HID), lambda i: (0, 0)),
        ],
        out_specs=pl.BlockSpec((BR, D_HID), lambda i: (i, 0)),
        out_shape=jax.ShapeDtypeStruct((N, D_HID), jnp.float32),
    )(h, p0, p1, d0, d1, W, b.reshape(1, D_HID))


def _mix_fc_body(h_ref, p0_ref, p1_ref, d0_ref, d1_ref, w_ref, b_ref,
                 wfc_ref, bfc_ref, o_ref):
    dsum = jnp.maximum(d0_ref[:, 0:1] + d1_ref[:, 0:1], 1.0)
    agg = (p0_ref[...] + p1_ref[...]) / dsum
    z = jnp.dot(h_ref[...] + agg, w_ref[...],
                preferred_element_type=jnp.float32)
    h2 = jnp.maximum(z + b_ref[...], 0.0)
    o_ref[...] = jnp.dot(h2, wfc_ref[...],
                         preferred_element_type=jnp.float32) + bfc_ref[...]


def _mix_fc(h, p0, p1, d0, d1, W, b, Wfc, bfc):
    return pl.pallas_call(
        _mix_fc_body,
        grid=(N // BR,),
        in_specs=[
            pl.BlockSpec((BR, D_HID), lambda i: (i, 0)),
            pl.BlockSpec((BR, D_HID), lambda i: (i, 0)),
            pl.BlockSpec((BR, D_HID), lambda i: (i, 0)),
            pl.BlockSpec((BR, D_HID), lambda i: (i, 0)),
            pl.BlockSpec((BR, D_HID), lambda i: (i, 0)),
            pl.BlockSpec((D_HID, D_HID), lambda i: (0, 0)),
            pl.BlockSpec((1, D_HID), lambda i: (0, 0)),
            pl.BlockSpec((D_HID, D_OUT), lambda i: (0, 0)),
            pl.BlockSpec((1, D_OUT), lambda i: (0, 0)),
        ],
        out_specs=pl.BlockSpec((BR, D_OUT), lambda i: (i, 0)),
        out_shape=jax.ShapeDtypeStruct((N, D_OUT), jnp.float32),
    )(h, p0, p1, d0, d1, W, b.reshape(1, D_HID), Wfc, bfc.reshape(1, D_OUT))


def kernel(x, edge_index, W_fe, b_fe, W_g1, b_g1, W_g2, b_g2, W_fc, b_fc):
    src = edge_index[0].astype(jnp.int32).reshape(NW, EPW)
    dst = edge_index[1].astype(jnp.int32).reshape(NW, EPW)
    srcf = src[:, :NCHF * C].reshape(NW, NCHF, C)
    dstf = dst[:, :NCHF * C].reshape(NW, NCHF, C)
    srct = src[:, NCHF * C:]
    dstt = dst[:, NCHF * C:]

    h0 = _fe(x, W_fe, b_fe)
    (degp,) = _deg(dstf, dstt)
    (p1,) = _agg(h0, srcf, dstf, srct, dstt)
    h1 = _mix(h0, p1[0], p1[1], degp[0], degp[1], W_g1, b_g1)
    (p2,) = _agg(h1, srcf, dstf, srct, dstt)
    return _mix_fc(h1, p2[0], p2[1], degp[0], degp[1], W_g2, b_g2, W_fc, b_fc)


# speedup vs baseline: 8.2333x; 1.0707x over previous
import jax
import jax.numpy as jnp
from jax import lax
from jax.experimental import pallas as pl
from jax.experimental.pallas import tpu as pltpu
from jax.experimental.pallas import tpu_sc as plsc

N = 10000
E = 320000
D_IN = 128
D_HID = 128
D_OUT = 64

NC = 2
NS = 16
NW = NC * NS
EPW = E // NW
C = 104
NCHF = 96
CT = EPW - NCHF * C
STRIDE = 624
WIN = 640
ZR = 32


def _zero_rows(zrow_v, tgt_sh, sid, sem):
    zero16 = jnp.zeros((16,), jnp.float32)

    def zr_body(r, carry):
        for jj in range(D_HID // 16):
            zrow_v[r, pl.ds(jj * 16, 16)] = zero16
        return carry
    lax.fori_loop(0, ZR, zr_body, 0)
    for t in range(WIN // ZR):
        pltpu.async_copy(zrow_v, tgt_sh.at[pl.ds(sid * STRIDE + t * ZR, ZR)],
                         sem)
    for t in range(WIN // ZR):
        pltpu.make_async_copy(
            zrow_v, tgt_sh.at[pl.ds(sid * STRIDE + t * ZR, ZR)], sem).wait()


def _agg_body(h_hbm, srcf_hbm, dstf_hbm, srct_hbm, dstt_hbm, part_hbm,
              srcb_v, dstb_v, rows_v, srct_v, dstt_v, rowst_v, zrow_v,
              acc_sh, sg0, sg1, si0, si1):
    cid = lax.axis_index("c")
    sid = lax.axis_index("s")
    w = cid * NS + sid
    sg = (sg0, sg1)
    si = (si0, si1)

    _zero_rows(zrow_v, acc_sh, sid, si0)
    plsc.subcore_barrier()

    pltpu.sync_copy(srcf_hbm.at[w, 0], srcb_v.at[0])
    pltpu.sync_copy(dstf_hbm.at[w, 0], dstb_v.at[0])
    pltpu.async_copy(h_hbm.at[srcb_v.at[0]], rows_v.at[0], sg[0])
    pltpu.async_copy(srcf_hbm.at[w, 1], srcb_v.at[1], si[1])
    pltpu.async_copy(dstf_hbm.at[w, 1], dstb_v.at[1], si[1])

    def outer(jj, carry):
        for b in range(2):
            ob = 1 - b
            j = 2 * jj + b
            j1 = jnp.minimum(j + 1, NCHF - 1)
            j2 = jnp.minimum(j + 2, NCHF - 1)
            pltpu.make_async_copy(h_hbm.at[srcb_v.at[b]], rows_v.at[b],
                                  sg[b]).wait()
            pltpu.make_async_copy(srcf_hbm.at[w, j1], srcb_v.at[ob],
                                  si[ob]).wait()
            pltpu.make_async_copy(dstf_hbm.at[w, j1], dstb_v.at[ob],
                                  si[ob]).wait()
            pltpu.async_copy(h_hbm.at[srcb_v.at[ob]], rows_v.at[ob], sg[ob])
            pltpu.sync_copy(rows_v.at[b], acc_sh.at[dstb_v.at[b]], add=True)
            pltpu.async_copy(srcf_hbm.at[w, j2], srcb_v.at[b], si[b])
            pltpu.async_copy(dstf_hbm.at[w, j2], dstb_v.at[b], si[b])
        return carry
    lax.fori_loop(0, NCHF // 2, outer, 0)

    pltpu.make_async_copy(h_hbm.at[srcb_v.at[0]], rows_v.at[0], sg[0]).wait()
    pltpu.make_async_copy(srcf_hbm.at[w, NCHF - 1], srcb_v.at[1],
                          si[1]).wait()
    pltpu.make_async_copy(dstf_hbm.at[w, NCHF - 1], dstb_v.at[1],
                          si[1]).wait()

    pltpu.sync_copy(srct_hbm.at[w], srct_v)
    pltpu.sync_copy(dstt_hbm.at[w], dstt_v)
    pltpu.async_copy(h_hbm.at[srct_v], rowst_v, sg[0]).wait()
    pltpu.sync_copy(rowst_v, acc_sh.at[dstt_v], add=True)

    plsc.subcore_barrier()

    pltpu.sync_copy(acc_sh.at[pl.ds(sid * STRIDE, WIN)],
                    part_hbm.at[cid, pl.ds(sid * STRIDE, WIN)])


def _deg_body(dstf_hbm, dstt_hbm, deg_hbm, dstb_v, dstt_v, ones_v, zrow_v,
              deg_sh, si0, si1):
    cid = lax.axis_index("c")
    sid = lax.axis_index("s")
    w = cid * NS + sid
    si = (si0, si1)

    one16 = jnp.ones((16,), jnp.float32)

    def on_body(r, carry):
        for jj in range(D_HID // 16):
            ones_v[r, pl.ds(jj * 16, 16)] = one16
        return carry
    lax.fori_loop(0, C, on_body, 0)

    _zero_rows(zrow_v, deg_sh, sid, si0)
    plsc.subcore_barrier()

    pltpu.sync_copy(dstf_hbm.at[w, 0], dstb_v.at[0])
    pltpu.async_copy(dstf_hbm.at[w, 1], dstb_v.at[1], si[1])

    def outer(jj, carry):
        for b in range(2):
            ob = 1 - b
            j = 2 * jj + b
            j1 = jnp.minimum(j + 1, NCHF - 1)
            pltpu.sync_copy(ones_v, deg_sh.at[dstb_v.at[b]], add=True)
            pltpu.make_async_copy(dstf_hbm.at[w, j1], dstb_v.at[ob],
                                  si[ob]).wait()
            pltpu.async_copy(dstf_hbm.at[w, jnp.minimum(j + 2, NCHF - 1)],
                             dstb_v.at[b], si[b])
        return carry
    lax.fori_loop(0, NCHF // 2, outer, 0)

    pltpu.make_async_copy(dstf_hbm.at[w, NCHF - 1], dstb_v.at[1],
                          si[1]).wait()

    pltpu.sync_copy(dstt_hbm.at[w], dstt_v)
    pltpu.sync_copy(ones_v.at[pl.ds(0, CT)], deg_sh.at[dstt_v], add=True)

    plsc.subcore_barrier()

    pltpu.sync_copy(deg_sh.at[pl.ds(sid * STRIDE, WIN)],
                    deg_hbm.at[cid, pl.ds(sid * STRIDE, WIN)])


_SC_MESH = plsc.VectorSubcoreMesh(core_axis_name="c", subcore_axis_name="s")

_agg = pl.kernel(
    _agg_body,
    out_type=[jax.ShapeDtypeStruct((NC, N, D_HID), jnp.float32)],
    mesh=_SC_MESH,
    scratch_types=[
        pltpu.VMEM((2, C), jnp.int32),
        pltpu.VMEM((2, C), jnp.int32),
        pltpu.VMEM((2, C, D_HID), jnp.float32),
        pltpu.VMEM((CT,), jnp.int32),
        pltpu.VMEM((CT,), jnp.int32),
        pltpu.VMEM((CT, D_HID), jnp.float32),
        pltpu.VMEM((ZR, D_HID), jnp.float32),
        pltpu.VMEM_SHARED((N, D_HID), jnp.float32),
        pltpu.SemaphoreType.DMA,
        pltpu.SemaphoreType.DMA,
        pltpu.SemaphoreType.DMA,
        pltpu.SemaphoreType.DMA,
    ],
)

_deg = pl.kernel(
    _deg_body,
    out_type=[jax.ShapeDtypeStruct((NC, N, D_HID), jnp.float32)],
    mesh=_SC_MESH,
    scratch_types=[
        pltpu.VMEM((2, C), jnp.int32),
        pltpu.VMEM((CT,), jnp.int32),
        pltpu.VMEM((C, D_HID), jnp.float32),
        pltpu.VMEM((ZR, D_HID), jnp.float32),
        pltpu.VMEM_SHARED((N, D_HID), jnp.float32),
        pltpu.SemaphoreType.DMA,
        pltpu.SemaphoreType.DMA,
    ],
)


BR = 1000


def _fe_body(x_ref, w_ref, b_ref, o_ref):
    z = jnp.dot(x_ref[...], w_ref[...], preferred_element_type=jnp.float32)
    o_ref[...] = jnp.tanh(z + b_ref[...])


def _fe(x, W, b):
    return pl.pallas_call(
        _fe_body,
        grid=(N // BR,),
        in_specs=[
            pl.BlockSpec((BR, D_IN), lambda i: (i, 0)),
            pl.BlockSpec((D_IN, D_HID), lambda i: (0, 0)),
            pl.BlockSpec((1, D_HID), lambda i: (0, 0)),
        ],
        out_specs=pl.BlockSpec((BR, D_HID), lambda i: (i, 0)),
        out_shape=jax.ShapeDtypeStruct((N, D_HID), jnp.float32),
    )(x, W, b.reshape(1, D_HID))


_P_SPECS = [
    pl.BlockSpec((1, BR, D_HID), lambda i: (0, i, 0)),
    pl.BlockSpec((1, BR, D_HID), lambda i: (1, i, 0)),
]


def _mix_body(h_ref, p0_ref, p1_ref, d0_ref, d1_ref, w_ref, b_ref, o_ref):
    dsum = jnp.maximum(d0_ref[0, :, 0:1] + d1_ref[0, :, 0:1], 1.0)
    agg = (p0_ref[0] + p1_ref[0]) / dsum
    z = jnp.dot(h_ref[...] + agg, w_ref[...],
                preferred_element_type=jnp.float32)
    o_ref[...] = jnp.maximum(z + b_ref[...], 0.0)


def _mix(h, part, degp, W, b):
    return pl.pallas_call(
        _mix_body,
        grid=(N // BR,),
        in_specs=[
            pl.BlockSpec((BR, D_HID), lambda i: (i, 0)),
            _P_SPECS[0], _P_SPECS[1], _P_SPECS[0], _P_SPECS[1],
            pl.BlockSpec((D_HID, D_HID), lambda i: (0, 0)),
            pl.BlockSpec((1, D_HID), lambda i: (0, 0)),
        ],
        out_specs=pl.BlockSpec((BR, D_HID), lambda i: (i, 0)),
        out_shape=jax.ShapeDtypeStruct((N, D_HID), jnp.float32),
    )(h, part, part, degp, degp, W, b.reshape(1, D_HID))


def _mix_fc_body(h_ref, p0_ref, p1_ref, d0_ref, d1_ref, w_ref, b_ref,
                 wfc_ref, bfc_ref, o_ref):
    dsum = jnp.maximum(d0_ref[0, :, 0:1] + d1_ref[0, :, 0:1], 1.0)
    agg = (p0_ref[0] + p1_ref[0]) / dsum
    z = jnp.dot(h_ref[...] + agg, w_ref[...],
                preferred_element_type=jnp.float32)
    h2 = jnp.maximum(z + b_ref[...], 0.0)
    o_ref[...] = jnp.dot(h2, wfc_ref[...],
                         preferred_element_type=jnp.float32) + bfc_ref[...]


def _mix_fc(h, part, degp, W, b, Wfc, bfc):
    return pl.pallas_call(
        _mix_fc_body,
        grid=(N // BR,),
        in_specs=[
            pl.BlockSpec((BR, D_HID), lambda i: (i, 0)),
            _P_SPECS[0], _P_SPECS[1], _P_SPECS[0], _P_SPECS[1],
            pl.BlockSpec((D_HID, D_HID), lambda i: (0, 0)),
            pl.BlockSpec((1, D_HID), lambda i: (0, 0)),
            pl.BlockSpec((D_HID, D_OUT), lambda i: (0, 0)),
            pl.BlockSpec((1, D_OUT), lambda i: (0, 0)),
        ],
        out_specs=pl.BlockSpec((BR, D_OUT), lambda i: (i, 0)),
        out_shape=jax.ShapeDtypeStruct((N, D_OUT), jnp.float32),
    )(h, part, part, degp, degp, W, b.reshape(1, D_HID),
      Wfc, bfc.reshape(1, D_OUT))


def kernel(x, edge_index, W_fe, b_fe, W_g1, b_g1, W_g2, b_g2, W_fc, b_fc):
    src = edge_index[0].astype(jnp.int32).reshape(NW, EPW)
    dst = edge_index[1].astype(jnp.int32).reshape(NW, EPW)
    srcf = src[:, :NCHF * C].reshape(NW, NCHF, C)
    dstf = dst[:, :NCHF * C].reshape(NW, NCHF, C)
    srct = src[:, NCHF * C:]
    dstt = dst[:, NCHF * C:]

    h0 = _fe(x, W_fe, b_fe)
    (degp,) = _deg(dstf, dstt)
    (p1,) = _agg(h0, srcf, dstf, srct, dstt)
    h1 = _mix(h0, p1, degp, W_g1, b_g1)
    (p2,) = _agg(h1, srcf, dstf, srct, dstt)
    return _mix_fc(h1, p2, degp, W_g2, b_g2, W_fc, b_fc)


# speedup vs baseline: 8.2968x; 1.0077x over previous
import jax
import jax.numpy as jnp
from jax import lax
from jax.experimental import pallas as pl
from jax.experimental.pallas import tpu as pltpu
from jax.experimental.pallas import tpu_sc as plsc

N = 10000
E = 320000
D_IN = 128
D_HID = 128
D_OUT = 64

NC = 2
NS = 16
NW = NC * NS
EPW = E // NW
C = 104
NCHF = 96
CT = EPW - NCHF * C
G = 8
NG = NCHF // G
STRIDE = 624
WIN = 640
ZR = 32


def _zero_rows(zrow_v, tgt_sh, sid, sem):
    zero16 = jnp.zeros((16,), jnp.float32)

    def zr_body(r, carry):
        for jj in range(D_HID // 16):
            zrow_v[r, pl.ds(jj * 16, 16)] = zero16
        return carry
    lax.fori_loop(0, ZR, zr_body, 0)
    for t in range(WIN // ZR):
        pltpu.async_copy(zrow_v, tgt_sh.at[pl.ds(sid * STRIDE + t * ZR, ZR)],
                         sem)
    for t in range(WIN // ZR):
        pltpu.make_async_copy(
            zrow_v, tgt_sh.at[pl.ds(sid * STRIDE + t * ZR, ZR)], sem).wait()


def _agg_body(h_hbm, srcf_hbm, dstf_hbm, srct_hbm, dstt_hbm, part_hbm,
              srcb_v, dstb_v, rows_v, srct_v, dstt_v, rowst_v, zrow_v,
              acc_sh, sg0, sg1, si0, si1):
    cid = lax.axis_index("c")
    sid = lax.axis_index("s")
    w = cid * NS + sid
    sg = (sg0, sg1)
    si = (si0, si1)

    _zero_rows(zrow_v, acc_sh, sid, si0)
    plsc.subcore_barrier()

    pltpu.async_copy(srcf_hbm.at[w, pl.ds(0, G)], srcb_v.at[0], si[0])
    pltpu.async_copy(dstf_hbm.at[w, pl.ds(0, G)], dstb_v.at[0], si[0])
    pltpu.async_copy(srcf_hbm.at[w, pl.ds(G, G)], srcb_v.at[1], si[1])
    pltpu.async_copy(dstf_hbm.at[w, pl.ds(G, G)], dstb_v.at[1], si[1])
    pltpu.make_async_copy(srcf_hbm.at[w, pl.ds(0, G)], srcb_v.at[0],
                          si[0]).wait()
    pltpu.make_async_copy(dstf_hbm.at[w, pl.ds(0, G)], dstb_v.at[0],
                          si[0]).wait()
    pltpu.async_copy(h_hbm.at[srcb_v.at[0, 0]], rows_v.at[0], sg[0])

    def outer(gp, carry):
        for gs in range(2):
            ns = 1 - gs
            gg = 2 * gp + gs
            for k in range(G):
                b = k % 2
                ob = 1 - b
                pltpu.make_async_copy(h_hbm.at[srcb_v.at[gs, k]],
                                      rows_v.at[b], sg[b]).wait()
                if k < G - 1:
                    pltpu.async_copy(h_hbm.at[srcb_v.at[gs, k + 1]],
                                     rows_v.at[ob], sg[ob])
                else:
                    pltpu.make_async_copy(srcf_hbm.at[w, pl.ds(0, G)],
                                          srcb_v.at[ns], si[ns]).wait()
                    pltpu.make_async_copy(dstf_hbm.at[w, pl.ds(0, G)],
                                          dstb_v.at[ns], si[ns]).wait()
                    pltpu.async_copy(h_hbm.at[srcb_v.at[ns, 0]],
                                     rows_v.at[ob], sg[ob])
                pltpu.sync_copy(rows_v.at[b], acc_sh.at[dstb_v.at[gs, k]],
                                add=True)
            g2 = jnp.minimum(gg + 2, NG - 1) * G
            pltpu.async_copy(srcf_hbm.at[w, pl.ds(g2, G)], srcb_v.at[gs],
                             si[gs])
            pltpu.async_copy(dstf_hbm.at[w, pl.ds(g2, G)], dstb_v.at[gs],
                             si[gs])
        return carry
    lax.fori_loop(0, NG // 2, outer, 0)

    pltpu.make_async_copy(h_hbm.at[srcb_v.at[0, 0]], rows_v.at[0],
                          sg[0]).wait()
    pltpu.make_async_copy(srcf_hbm.at[w, pl.ds(0, G)], srcb_v.at[1],
                          si[1]).wait()
    pltpu.make_async_copy(dstf_hbm.at[w, pl.ds(0, G)], dstb_v.at[1],
                          si[1]).wait()

    pltpu.sync_copy(srct_hbm.at[w], srct_v)
    pltpu.sync_copy(dstt_hbm.at[w], dstt_v)
    pltpu.async_copy(h_hbm.at[srct_v], rowst_v, sg[0]).wait()
    pltpu.sync_copy(rowst_v, acc_sh.at[dstt_v], add=True)

    plsc.subcore_barrier()

    pltpu.sync_copy(acc_sh.at[pl.ds(sid * STRIDE, WIN)],
                    part_hbm.at[cid, pl.ds(sid * STRIDE, WIN)])


def _deg_body(dstf_hbm, dstt_hbm, deg_hbm, dstb_v, dstt_v, ones_v, zrow_v,
              deg_sh, si0, si1):
    cid = lax.axis_index("c")
    sid = lax.axis_index("s")
    w = cid * NS + sid
    si = (si0, si1)

    one16 = jnp.ones((16,), jnp.float32)

    def on_body(r, carry):
        for jj in range(D_HID // 16):
            ones_v[r, pl.ds(jj * 16, 16)] = one16
        return carry
    lax.fori_loop(0, C, on_body, 0)

    _zero_rows(zrow_v, deg_sh, sid, si0)
    plsc.subcore_barrier()

    pltpu.sync_copy(dstf_hbm.at[w, 0], dstb_v.at[0])
    pltpu.async_copy(dstf_hbm.at[w, 1], dstb_v.at[1], si[1])

    def outer(jj, carry):
        for b in range(2):
            ob = 1 - b
            j = 2 * jj + b
            j1 = jnp.minimum(j + 1, NCHF - 1)
            pltpu.sync_copy(ones_v, deg_sh.at[dstb_v.at[b]], add=True)
            pltpu.make_async_copy(dstf_hbm.at[w, j1], dstb_v.at[ob],
                                  si[ob]).wait()
            pltpu.async_copy(dstf_hbm.at[w, jnp.minimum(j + 2, NCHF - 1)],
                             dstb_v.at[b], si[b])
        return carry
    lax.fori_loop(0, NCHF // 2, outer, 0)

    pltpu.make_async_copy(dstf_hbm.at[w, NCHF - 1], dstb_v.at[1],
                          si[1]).wait()

    pltpu.sync_copy(dstt_hbm.at[w], dstt_v)
    pltpu.sync_copy(ones_v.at[pl.ds(0, CT)], deg_sh.at[dstt_v], add=True)

    plsc.subcore_barrier()

    pltpu.sync_copy(deg_sh.at[pl.ds(sid * STRIDE, WIN)],
                    deg_hbm.at[cid, pl.ds(sid * STRIDE, WIN)])


_SC_MESH = plsc.VectorSubcoreMesh(core_axis_name="c", subcore_axis_name="s")

_agg = pl.kernel(
    _agg_body,
    out_type=[jax.ShapeDtypeStruct((NC, N, D_HID), jnp.float32)],
    mesh=_SC_MESH,
    scratch_types=[
        pltpu.VMEM((2, G, C), jnp.int32),
        pltpu.VMEM((2, G, C), jnp.int32),
        pltpu.VMEM((2, C, D_HID), jnp.float32),
        pltpu.VMEM((CT,), jnp.int32),
        pltpu.VMEM((CT,), jnp.int32),
        pltpu.VMEM((CT, D_HID), jnp.float32),
        pltpu.VMEM((ZR, D_HID), jnp.float32),
        pltpu.VMEM_SHARED((N, D_HID), jnp.float32),
        pltpu.SemaphoreType.DMA,
        pltpu.SemaphoreType.DMA,
        pltpu.SemaphoreType.DMA,
        pltpu.SemaphoreType.DMA,
    ],
)

_deg = pl.kernel(
    _deg_body,
    out_type=[jax.ShapeDtypeStruct((NC, N, D_HID), jnp.float32)],
    mesh=_SC_MESH,
    scratch_types=[
        pltpu.VMEM((2, C), jnp.int32),
        pltpu.VMEM((CT,), jnp.int32),
        pltpu.VMEM((C, D_HID), jnp.float32),
        pltpu.VMEM((ZR, D_HID), jnp.float32),
        pltpu.VMEM_SHARED((N, D_HID), jnp.float32),
        pltpu.SemaphoreType.DMA,
        pltpu.SemaphoreType.DMA,
    ],
)


BR = 1000


def _fe_body(x_ref, w_ref, b_ref, o_ref):
    z = jnp.dot(x_ref[...], w_ref[...], preferred_element_type=jnp.float32)
    o_ref[...] = jnp.tanh(z + b_ref[...])


def _fe(x, W, b):
    return pl.pallas_call(
        _fe_body,
        grid=(N // BR,),
        in_specs=[
            pl.BlockSpec((BR, D_IN), lambda i: (i, 0)),
            pl.BlockSpec((D_IN, D_HID), lambda i: (0, 0)),
            pl.BlockSpec((1, D_HID), lambda i: (0, 0)),
        ],
        out_specs=pl.BlockSpec((BR, D_HID), lambda i: (i, 0)),
        out_shape=jax.ShapeDtypeStruct((N, D_HID), jnp.float32),
    )(x, W, b.reshape(1, D_HID))


_P_SPECS = [
    pl.BlockSpec((1, BR, D_HID), lambda i: (0, i, 0)),
    pl.BlockSpec((1, BR, D_HID), lambda i: (1, i, 0)),
]


def _mix_body(h_ref, p0_ref, p1_ref, d0_ref, d1_ref, w_ref, b_ref, o_ref):
    dsum = jnp.maximum(d0_ref[0, :, 0:1] + d1_ref[0, :, 0:1], 1.0)
    agg = (p0_ref[0] + p1_ref[0]) / dsum
    z = jnp.dot(h_ref[...] + agg, w_ref[...],
                preferred_element_type=jnp.float32)
    o_ref[...] = jnp.maximum(z + b_ref[...], 0.0)


def _mix(h, part, degp, W, b):
    return pl.pallas_call(
        _mix_body,
        grid=(N // BR,),
        in_specs=[
            pl.BlockSpec((BR, D_HID), lambda i: (i, 0)),
            _P_SPECS[0], _P_SPECS[1], _P_SPECS[0], _P_SPECS[1],
            pl.BlockSpec((D_HID, D_HID), lambda i: (0, 0)),
            pl.BlockSpec((1, D_HID), lambda i: (0, 0)),
        ],
        out_specs=pl.BlockSpec((BR, D_HID), lambda i: (i, 0)),
        out_shape=jax.ShapeDtypeStruct((N, D_HID), jnp.float32),
    )(h, part, part, degp, degp, W, b.reshape(1, D_HID))


def _mix_fc_body(h_ref, p0_ref, p1_ref, d0_ref, d1_ref, w_ref, b_ref,
                 wfc_ref, bfc_ref, o_ref):
    dsum = jnp.maximum(d0_ref[0, :, 0:1] + d1_ref[0, :, 0:1], 1.0)
    agg = (p0_ref[0] + p1_ref[0]) / dsum
    z = jnp.dot(h_ref[...] + agg, w_ref[...],
                preferred_element_type=jnp.float32)
    h2 = jnp.maximum(z + b_ref[...], 0.0)
    o_ref[...] = jnp.dot(h2, wfc_ref[...],
                         preferred_element_type=jnp.float32) + bfc_ref[...]


def _mix_fc(h, part, degp, W, b, Wfc, bfc):
    return pl.pallas_call(
        _mix_fc_body,
        grid=(N // BR,),
        in_specs=[
            pl.BlockSpec((BR, D_HID), lambda i: (i, 0)),
            _P_SPECS[0], _P_SPECS[1], _P_SPECS[0], _P_SPECS[1],
            pl.BlockSpec((D_HID, D_HID), lambda i: (0, 0)),
            pl.BlockSpec((1, D_HID), lambda i: (0, 0)),
            pl.BlockSpec((D_HID, D_OUT), lambda i: (0, 0)),
            pl.BlockSpec((1, D_OUT), lambda i: (0, 0)),
        ],
        out_specs=pl.BlockSpec((BR, D_OUT), lambda i: (i, 0)),
        out_shape=jax.ShapeDtypeStruct((N, D_OUT), jnp.float32),
    )(h, part, part, degp, degp, W, b.reshape(1, D_HID),
      Wfc, bfc.reshape(1, D_OUT))


def kernel(x, edge_index, W_fe, b_fe, W_g1, b_g1, W_g2, b_g2, W_fc, b_fc):
    src = edge_index[0].astype(jnp.int32).reshape(NW, EPW)
    dst = edge_index[1].astype(jnp.int32).reshape(NW, EPW)
    srcf = src[:, :NCHF * C].reshape(NW, NCHF, C)
    dstf = dst[:, :NCHF * C].reshape(NW, NCHF, C)
    srct = src[:, NCHF * C:]
    dstt = dst[:, NCHF * C:]

    h0 = _fe(x, W_fe, b_fe)
    (degp,) = _deg(dstf, dstt)
    (p1,) = _agg(h0, srcf, dstf, srct, dstt)
    h1 = _mix(h0, p1, degp, W_g1, b_g1)
    (p2,) = _agg(h1, srcf, dstf, srct, dstt)
    return _mix_fc(h1, p2, degp, W_g2, b_g2, W_fc, b_fc)
